# baseline (speedup 1.0000x reference)
import functools

import jax
import jax.numpy as jnp
from jax import lax
from jax.experimental import pallas as pl
from jax.experimental.pallas import tpu as pltpu
from jax.experimental.pallas import tpu_sc as plsc

_NUM_CORES = 2
_NUM_SUBCORES = 16
_NUM_WORKERS = _NUM_CORES * _NUM_SUBCORES
_LANES = 16

_STREAM_ROWS = 128


def _make_sc_embed(l_dim, b_dim, n_sub, dim):
  groups = l_dim * b_dim
  assert groups % _NUM_WORKERS == 0
  g_per_w = groups // _NUM_WORKERS

  g_chunk = 256
  while g_per_w % g_chunk != 0 or b_dim % g_chunk != 0:
    g_chunk //= 2
  r_chunk = g_chunk * n_sub
  n_steps = g_per_w // g_chunk
  n_streams = pl.cdiv(g_chunk, _STREAM_ROWS)
  n_vb = g_chunk // _LANES

  mesh = plsc.VectorSubcoreMesh(core_axis_name="c", subcore_axis_name="s")

  @functools.partial(
      pl.kernel,
      out_type=jax.ShapeDtypeStruct((l_dim, dim, b_dim), jnp.float32),
      mesh=mesh,
      scratch_types=[
          pltpu.VMEM((n_sub, g_chunk), jnp.int32),
          pltpu.VMEM((r_chunk, dim), jnp.float32),
          pltpu.VMEM((dim, g_chunk), jnp.float32),
          pltpu.VMEM((_LANES, dim), jnp.float32),
          pltpu.VMEM((dim, _LANES), jnp.float32),
          pltpu.VMEM((_LANES,), jnp.int32),
          pltpu.SemaphoreType.DMA,
          pltpu.SemaphoreType.DMA,
      ],
      compiler_params=pltpu.CompilerParams(
          use_tc_tiling_on_sc=False, needs_layout_passes=False),
  )
  def k(tt_hbm, table_hbm, out_hbm, idx_v, rows_v, slab_v, row0x_v, row0b_v,
        zidx_v, gsem, osem):
    wid = lax.axis_index("s") * _NUM_CORES + lax.axis_index("c")
    g_base = wid * g_per_w
    lanes = lax.iota(jnp.int32, _LANES)
    zidx_v[...] = jnp.zeros((_LANES,), jnp.int32)
    pltpu.async_copy(table_hbm.at[zidx_v], row0x_v, gsem).wait()
    for d in range(dim):
      row0b_v[d, :] = plsc.load_gather(
          row0x_v, [lanes, jnp.full((_LANES,), d, jnp.int32)])

    def step(c, carry):
      del carry
      g0 = g_base + c * g_chunk
      l = g0 // b_dim
      b0 = g0 % b_dim
      pltpu.sync_copy(tt_hbm.at[l, :, pl.ds(b0, g_chunk)], idx_v)
      copies = []
      for n in range(n_sub):
        for s in range(n_streams):
          copies.append(pltpu.async_copy(
              table_hbm.at[idx_v.at[n, pl.ds(s * _STREAM_ROWS, _STREAM_ROWS)]],
              rows_v.at[pl.ds(n * g_chunk + s * _STREAM_ROWS, _STREAM_ROWS), :],
              gsem))
      for cp in copies:
        cp.wait()

      def vb_body(vb, carry2):
        del carry2
        col = pl.ds(vb * _LANES, _LANES)
        cnt = jnp.zeros((_LANES,), jnp.float32)
        for n in range(n_sub):
          cnt = cnt + jnp.where(idx_v[n, col] == 0, 1.0, 0.0)
        nonpad = float(n_sub) - cnt
        empty = nonpad == 0.0
        scale = jnp.where(empty, 0.0,
                          1.0 / jnp.where(empty, 1.0, nonpad))
        row_idx = [lanes + vb * _LANES + n * g_chunk for n in range(n_sub)]
        for d in range(dim):
          dvec = jnp.full((_LANES,), d, jnp.int32)
          acc = plsc.load_gather(rows_v, [row_idx[0], dvec])
          for n in range(1, n_sub):
            acc = acc + plsc.load_gather(rows_v, [row_idx[n], dvec])
          r0d = row0b_v[d, :]
          slab_v[d, col] = (acc - cnt * r0d) * scale
        return 0

      lax.fori_loop(0, n_vb, vb_body, 0)
      pltpu.async_copy(
          slab_v, out_hbm.at[l, :, pl.ds(b0, g_chunk)], osem).wait()
      return 0

    lax.fori_loop(0, n_steps, step, 0)

  return k


def kernel(token_ids, table):
  b, l, n_sub = token_ids.shape
  vocab, dim = table.shape

  tt = jnp.transpose(token_ids, (1, 2, 0))
  sc_embed = _make_sc_embed(l, b, n_sub, dim)
  out_t = sc_embed(tt, table)
  return jnp.transpose(out_t, (2, 0, 1))


# speedup vs baseline: 1.6272x; 1.6272x over previous
import functools

import jax
import jax.numpy as jnp
from jax import lax
from jax.experimental import pallas as pl
from jax.experimental.pallas import tpu as pltpu
from jax.experimental.pallas import tpu_sc as plsc

_NUM_CORES = 2
_NUM_SUBCORES = 16
_NUM_WORKERS = _NUM_CORES * _NUM_SUBCORES
_LANES = 16

_STREAM_ROWS = 128


def _make_sc_group_sum(l_dim, b_dim, n_sub, dim):
  groups = l_dim * b_dim
  assert groups % _NUM_WORKERS == 0
  g_per_w = groups // _NUM_WORKERS

  g_chunk = 256
  while g_per_w % g_chunk != 0 or b_dim % g_chunk != 0:
    g_chunk //= 2
  r_chunk = g_chunk * n_sub
  n_steps = g_per_w // g_chunk
  n_streams = pl.cdiv(g_chunk, _STREAM_ROWS)

  mesh = plsc.VectorSubcoreMesh(core_axis_name="c", subcore_axis_name="s")

  @functools.partial(
      pl.kernel,
      out_type=jax.ShapeDtypeStruct((l_dim, b_dim, dim), jnp.float32),
      mesh=mesh,
      scratch_types=[
          pltpu.VMEM((n_sub, g_chunk), jnp.int32),
          pltpu.VMEM((r_chunk, dim), jnp.float32),
          pltpu.VMEM((g_chunk, dim), jnp.float32),
          pltpu.SemaphoreType.DMA,
          pltpu.SemaphoreType.DMA,
      ],
      compiler_params=pltpu.CompilerParams(use_tc_tiling_on_sc=False),
  )
  def k(tt_hbm, table_hbm, out_hbm, idx_v, rows_v, sums_v, gsem, osem):
    wid = lax.axis_index("s") * _NUM_CORES + lax.axis_index("c")
    g_base = wid * g_per_w

    def step(c, carry):
      del carry
      g0 = g_base + c * g_chunk
      l = g0 // b_dim
      b0 = g0 % b_dim
      pltpu.sync_copy(tt_hbm.at[l, :, pl.ds(b0, g_chunk)], idx_v)
      copies = []
      for n in range(n_sub):
        for s in range(n_streams):
          copies.append(pltpu.async_copy(
              table_hbm.at[idx_v.at[n, pl.ds(s * _STREAM_ROWS, _STREAM_ROWS)]],
              rows_v.at[pl.ds(n * g_chunk + s * _STREAM_ROWS, _STREAM_ROWS), :],
              gsem))
      for cp in copies:
        cp.wait()

      def pool(g, carry2):
        del carry2
        for h in range(dim // _LANES):
          d = pl.ds(h * _LANES, _LANES)
          acc = rows_v[g, d]
          for n in range(1, n_sub):
            acc = acc + rows_v[n * g_chunk + g, d]
          sums_v[g, d] = acc
        return 0

      lax.fori_loop(0, g_chunk, pool, 0)
      pltpu.async_copy(
          sums_v, out_hbm.at[l, pl.ds(b0, g_chunk), :], osem).wait()
      return 0

    lax.fori_loop(0, n_steps, step, 0)

  return k


def _fixup_body(s_ref, tt_ref, row0_ref, o_ref, *, n_sub, dim):
  ids = tt_ref[...]
  npad = jnp.sum((ids == 0).astype(jnp.float32), axis=1, keepdims=True)
  n = float(n_sub) - npad
  empty = n == 0.0
  denom = jnp.where(empty, 1.0, n)
  st = jnp.swapaxes(s_ref[...], 1, 2)
  row0t = row0_ref[...].reshape(1, dim, 1)
  o_ref[...] = jnp.where(empty, 0.0, (st - npad * row0t) / denom)


def kernel(token_ids, table):
  b, l, n_sub = token_ids.shape
  vocab, dim = table.shape

  tt = jnp.transpose(token_ids, (1, 2, 0))
  sc_sum = _make_sc_group_sum(l, b, n_sub, dim)
  s = sc_sum(tt, table)

  row0 = lax.slice(table, (0, 0), (1, dim))

  l_blk = 8
  while l % l_blk != 0:
    l_blk //= 2
  out_t = pl.pallas_call(
      functools.partial(_fixup_body, n_sub=n_sub, dim=dim),
      grid=(l // l_blk,),
      in_specs=[
          pl.BlockSpec((l_blk, b, dim), lambda i: (i, 0, 0)),
          pl.BlockSpec((l_blk, n_sub, b), lambda i: (i, 0, 0)),
          pl.BlockSpec((1, dim), lambda i: (0, 0)),
      ],
      out_specs=pl.BlockSpec((l_blk, dim, b), lambda i: (i, 0, 0)),
      out_shape=jax.ShapeDtypeStruct((l, dim, b), jnp.float32),
  )(s, tt, row0)

  return jnp.transpose(out_t, (2, 0, 1))


# speedup vs baseline: 1.6297x; 1.0016x over previous
import functools

import jax
import jax.numpy as jnp
from jax import lax
from jax.experimental import pallas as pl
from jax.experimental.pallas import tpu as pltpu
from jax.experimental.pallas import tpu_sc as plsc

_NUM_CORES = 2
_NUM_SUBCORES = 16
_NUM_WORKERS = _NUM_CORES * _NUM_SUBCORES
_LANES = 16

_STREAM_ROWS = 128


def _make_sc_group_sum(l_dim, b_dim, n_sub, dim):
  groups = l_dim * b_dim
  assert groups % _NUM_WORKERS == 0
  g_per_w = groups // _NUM_WORKERS

  g_chunk = 256
  while g_per_w % g_chunk != 0 or b_dim % g_chunk != 0:
    g_chunk //= 2
  r_chunk = g_chunk * n_sub
  n_steps = g_per_w // g_chunk
  n_streams = pl.cdiv(g_chunk, _STREAM_ROWS)

  mesh = plsc.VectorSubcoreMesh(core_axis_name="c", subcore_axis_name="s")

  @functools.partial(
      pl.kernel,
      out_type=jax.ShapeDtypeStruct((l_dim, b_dim, dim), jnp.float32),
      mesh=mesh,
      scratch_types=[
          pltpu.VMEM((n_sub, g_chunk), jnp.int32),
          pltpu.VMEM((r_chunk, dim), jnp.float32),
          pltpu.VMEM((g_chunk, dim), jnp.float32),
          pltpu.SemaphoreType.DMA,
          pltpu.SemaphoreType.DMA,
      ],
      compiler_params=pltpu.CompilerParams(use_tc_tiling_on_sc=False),
  )
  def k(tt_hbm, table_hbm, out_hbm, idx_v, rows_v, sums_v, gsem, osem):
    wid = lax.axis_index("s") * _NUM_CORES + lax.axis_index("c")
    g_base = wid * g_per_w

    def step(c, carry):
      del carry
      g0 = g_base + c * g_chunk
      l = g0 // b_dim
      b0 = g0 % b_dim
      pltpu.sync_copy(tt_hbm.at[l, :, pl.ds(b0, g_chunk)], idx_v)
      copies = []
      for n in range(n_sub):
        for s in range(n_streams):
          copies.append(pltpu.async_copy(
              table_hbm.at[idx_v.at[n, pl.ds(s * _STREAM_ROWS, _STREAM_ROWS)]],
              rows_v.at[pl.ds(n * g_chunk + s * _STREAM_ROWS, _STREAM_ROWS), :],
              gsem))
      for cp in copies:
        cp.wait()

      def pool(g, carry2):
        del carry2
        for h in range(dim // _LANES):
          d = pl.ds(h * _LANES, _LANES)
          acc = rows_v[g, d]
          for n in range(1, n_sub):
            acc = acc + rows_v[n * g_chunk + g, d]
          sums_v[g, d] = acc
        return 0

      lax.fori_loop(0, g_chunk, pool, 0)
      pltpu.async_copy(
          sums_v, out_hbm.at[l, pl.ds(b0, g_chunk), :], osem).wait()
      return 0

    lax.fori_loop(0, n_steps, step, 0)

  return k


def _fixup_body(s_ref, tt_ref, row0_ref, o_ref, *, n_sub, dim):
  ids = tt_ref[...]
  npad = jnp.sum((ids == 0).astype(jnp.float32), axis=1, keepdims=True)
  n = float(n_sub) - npad
  empty = n == 0.0
  denom = jnp.where(empty, 1.0, n)
  eye = jnp.eye(s_ref.shape[1], dtype=jnp.float32)
  st = jnp.einsum("lbd,bc->ldc", s_ref[...], eye,
                  preferred_element_type=jnp.float32)
  row0t = row0_ref[...].reshape(1, dim, 1)
  o_ref[...] = jnp.where(empty, 0.0, (st - npad * row0t) / denom)


def kernel(token_ids, table):
  b, l, n_sub = token_ids.shape
  vocab, dim = table.shape

  tt = jnp.transpose(token_ids, (1, 2, 0))
  sc_sum = _make_sc_group_sum(l, b, n_sub, dim)
  s = sc_sum(tt, table)

  row0 = lax.slice(table, (0, 0), (1, dim))

  l_blk = 8
  while l % l_blk != 0:
    l_blk //= 2
  out_t = pl.pallas_call(
      functools.partial(_fixup_body, n_sub=n_sub, dim=dim),
      grid=(l // l_blk,),
      in_specs=[
          pl.BlockSpec((l_blk, b, dim), lambda i: (i, 0, 0)),
          pl.BlockSpec((l_blk, n_sub, b), lambda i: (i, 0, 0)),
          pl.BlockSpec((1, dim), lambda i: (0, 0)),
      ],
      out_specs=pl.BlockSpec((l_blk, dim, b), lambda i: (i, 0, 0)),
      out_shape=jax.ShapeDtypeStruct((l, dim, b), jnp.float32),
  )(s, tt, row0)

  return jnp.transpose(out_t, (2, 0, 1))
